# baseline (speedup 1.0000x reference)
import functools

import jax
import jax.numpy as jnp
from jax import lax
from jax.experimental import pallas as pl
from jax.experimental.pallas import tpu as pltpu
from jax.experimental.pallas import tpu_sc as plsc

_NU = 10000
_NI = 10000
_E = 320000
_D = 128

_LANES = 128
_ROWS = _E // _LANES
_NC, _NS = 2, 16
_NW = _NC * _NS
_RPT = -(-(-(-_ROWS // _NW)) // 8) * 8

_ROWS_PAD = _RPT * _NW
_NPAD = 10240
_SEG = _NPAD // _NS

_EPT = _E // _NS
_EHALF = _EPT // 2


def _sc_hist_body(idx_a, idx_b, out_a, out_b,
                  buf0, buf1, hist0, sem0, sem1):
    c = lax.axis_index("c")
    s = lax.axis_index("s")
    base = s * _EPT

    zeros16 = jnp.zeros((16,), jnp.float32)
    ones16 = jnp.ones((16,), jnp.float32)

    def _zero(i, carry):
        for k in range(8):
            hist0[pl.ds((i * 8 + k) * 16, 16)] = zeros16
        return carry

    def _accum(buf):
        def body(j, carry):
            for k in range(25):
                iv = buf[pl.ds((j * 25 + k) * 16, 16)]
                plsc.addupdate_scatter(hist0, [iv], ones16)
            return carry

        lax.fori_loop(0, _EHALF // 400, body, 0)

    def _run(idx, out):
        cp0 = pltpu.async_copy(idx.at[pl.ds(base, _EHALF)], buf0, sem0)
        cp1 = pltpu.async_copy(idx.at[pl.ds(base + _EHALF, _EHALF)],
                               buf1, sem1)
        lax.fori_loop(0, _NPAD // 128, _zero, 0)
        cp0.wait()
        _accum(buf0)
        cp1.wait()
        _accum(buf1)
        pltpu.sync_copy(hist0, out.at[s])

    @pl.when(c == 0)
    def _():
        _run(idx_a, out_a)

    @pl.when(c == 1)
    def _():
        _run(idx_b, out_b)


@functools.cache
def _sc_degree_hist():
    mesh = plsc.VectorSubcoreMesh(
        core_axis_name="c", subcore_axis_name="s",
        num_cores=_NC, num_subcores=_NS)
    return pl.kernel(
        _sc_hist_body,
        out_type=(
            jax.ShapeDtypeStruct((_NS, _NPAD), jnp.float32),
            jax.ShapeDtypeStruct((_NS, _NPAD), jnp.float32),
        ),
        mesh=mesh,
        compiler_params=pltpu.CompilerParams(needs_layout_passes=False),
        scratch_types=[
            pltpu.VMEM((_EHALF,), jnp.int32),
            pltpu.VMEM((_EHALF,), jnp.int32),
            pltpu.VMEM((_NPAD,), jnp.float32),
            pltpu.SemaphoreType.DMA,
            pltpu.SemaphoreType.DMA,
        ],
    )


def _tc_body(ue_r, ie_r, ue_c, ie_c, wc_t, bc_c, wcb_t, bcb_c,
             wnu_h, wnu_m, bnu_r, wni_h, wni_m, bni_r, p_a, p_b,
             out_u, out_i, rows_v):
    @pl.when(pl.program_id(0) == 0)
    def _small():
        m_c = jax.nn.relu(
            jnp.sum(wc_t[...] * ue_r[...], axis=1, keepdims=True) + bc_c[...])
        m_cb = jax.nn.relu(
            jnp.sum(wcb_t[...] * ie_r[...], axis=1, keepdims=True)
            + bcb_c[...])
        rows_v[0:1, :] = (
            jnp.sum(ie_c[...] * wni_h[...], axis=0, keepdims=True)
            + bni_r[...])
        rows_v[1:2, :] = jnp.sum(m_c * wni_m[...], axis=0, keepdims=True)
        rows_v[2:3, :] = (
            jnp.sum(ue_c[...] * wnu_h[...], axis=0, keepdims=True)
            + bnu_r[...])
        rows_v[3:4, :] = jnp.sum(m_cb * wnu_m[...], axis=0, keepdims=True)

    deg_a = jnp.sum(p_a[...], axis=0, keepdims=True)
    deg_b = jnp.sum(p_b[...], axis=0, keepdims=True)
    dn = (((0,), (0,)), ((), ()))
    out_i[...] = jax.nn.relu(
        rows_v[0:1, :] + lax.dot_general(
            deg_a, rows_v[1:2, :], dn, preferred_element_type=jnp.float32))
    out_u[...] = jax.nn.relu(
        rows_v[2:3, :] + lax.dot_general(
            deg_b, rows_v[3:4, :], dn, preferred_element_type=jnp.float32))


_BLK = 1280


def _const_spec(shape):
    return pl.BlockSpec(shape, lambda i: (0,) * len(shape))


_tc_call = pl.pallas_call(
    _tc_body,
    grid=(_NPAD // _BLK,),
    in_specs=[
        _const_spec((1, _D)), _const_spec((1, _D)),
        _const_spec((_D, 1)), _const_spec((_D, 1)),
        _const_spec((_D, _D)), _const_spec((_D, 1)),
        _const_spec((_D, _D)), _const_spec((_D, 1)),
        _const_spec((_D, _D)), _const_spec((_D, _D)), _const_spec((1, _D)),
        _const_spec((_D, _D)), _const_spec((_D, _D)), _const_spec((1, _D)),
        pl.BlockSpec((2 * _NS, _BLK), lambda i: (0, i)),
        pl.BlockSpec((2 * _NS, _BLK), lambda i: (0, i)),
    ],
    out_specs=(
        pl.BlockSpec((_BLK, _D), lambda i: (i, 0)),
        pl.BlockSpec((_BLK, _D), lambda i: (i, 0)),
    ),
    out_shape=(
        jax.ShapeDtypeStruct((_NU, _D), jnp.float32),
        jax.ShapeDtypeStruct((_NI, _D), jnp.float32),
    ),
    scratch_shapes=[pltpu.VMEM((8, _D), jnp.float32)],
)


@jax.jit
def kernel(clicks_src, clicks_dst, clicked_by_src, clicked_by_dst,
           user_emb, item_emb,
           W_clicks, b_clicks, W_clicked_by, b_clicked_by,
           Wn_user, bn_user, Wn_item, bn_item):
    del clicks_src, clicked_by_src
    p_a, p_b = _sc_degree_hist()(clicks_dst.astype(jnp.int32),
                                 clicked_by_dst.astype(jnp.int32))

    out_u, out_i = _tc_call(
        user_emb, item_emb,
        user_emb.reshape(_D, 1), item_emb.reshape(_D, 1),
        W_clicks.T, b_clicks.reshape(_D, 1),
        W_clicked_by.T, b_clicked_by.reshape(_D, 1),
        Wn_user[:_D], Wn_user[_D:], bn_user.reshape(1, _D),
        Wn_item[:_D], Wn_item[_D:], bn_item.reshape(1, _D),
        p_a, p_b)
    return (out_u, out_i)


# speedup vs baseline: 1.0456x; 1.0456x over previous
import functools

import jax
import jax.numpy as jnp
from jax import lax
from jax.experimental import pallas as pl
from jax.experimental.pallas import tpu as pltpu
from jax.experimental.pallas import tpu_sc as plsc

_NU = 10000
_NI = 10000
_E = 320000
_D = 128

_LANES = 128
_ROWS = _E // _LANES
_NC, _NS = 2, 16
_NW = _NC * _NS
_RPT = -(-(-(-_ROWS // _NW)) // 8) * 8

_ROWS_PAD = _RPT * _NW
_NPAD = 10240
_SEG = _NPAD // _NS

_EPT = _E // _NS
_EHALF = _EPT // 2


def _sc_hist_body(idx_a, idx_b, out_a, out_b,
                  buf0, buf1, hist0, sem0, sem1):
    c = lax.axis_index("c")
    s = lax.axis_index("s")
    base = s * _EPT

    zeros16 = jnp.zeros((16,), jnp.float32)
    ones16 = jnp.ones((16,), jnp.float32)

    def _zero(i, carry):
        for k in range(8):
            hist0[pl.ds((i * 8 + k) * 16, 16)] = zeros16
        return carry

    def _accum(buf):
        def body(j, carry):
            for k in range(25):
                iv = buf[pl.ds((j * 25 + k) * 16, 16)]
                plsc.addupdate_scatter(hist0, [iv], ones16)
            return carry

        lax.fori_loop(0, _EHALF // 400, body, 0)

    def _run(idx, out):
        cp0 = pltpu.async_copy(idx.at[pl.ds(base, _EHALF)], buf0, sem0)
        cp1 = pltpu.async_copy(idx.at[pl.ds(base + _EHALF, _EHALF)],
                               buf1, sem1)
        lax.fori_loop(0, _NPAD // 128, _zero, 0)
        cp0.wait()
        _accum(buf0)
        cp1.wait()
        _accum(buf1)
        pltpu.sync_copy(hist0, out.at[s])

    @pl.when(c == 0)
    def _():
        _run(idx_a, out_a)

    @pl.when(c == 1)
    def _():
        _run(idx_b, out_b)


@functools.cache
def _sc_degree_hist():
    mesh = plsc.VectorSubcoreMesh(
        core_axis_name="c", subcore_axis_name="s",
        num_cores=_NC, num_subcores=_NS)
    return pl.kernel(
        _sc_hist_body,
        out_type=(
            jax.ShapeDtypeStruct((_NS, _NPAD), jnp.float32),
            jax.ShapeDtypeStruct((_NS, _NPAD), jnp.float32),
        ),
        mesh=mesh,
        compiler_params=pltpu.CompilerParams(needs_layout_passes=False),
        scratch_types=[
            pltpu.VMEM((_EHALF,), jnp.int32),
            pltpu.VMEM((_EHALF,), jnp.int32),
            pltpu.VMEM((_NPAD,), jnp.float32),
            pltpu.SemaphoreType.DMA,
            pltpu.SemaphoreType.DMA,
        ],
    )


def _tc_body(ue_r, ie_r, ue_c, ie_c, wc_t, bc_c, wcb_t, bcb_c,
             wnu_h, wnu_m, bnu_r, wni_h, wni_m, bni_r, p_a, p_b,
             out_u, out_i):
    m_c = jax.nn.relu(
        jnp.sum(wc_t[...] * ue_r[...], axis=1, keepdims=True) + bc_c[...])
    m_cb = jax.nn.relu(
        jnp.sum(wcb_t[...] * ie_r[...], axis=1, keepdims=True) + bcb_c[...])
    a_i = jnp.sum(ie_c[...] * wni_h[...], axis=0, keepdims=True) + bni_r[...]
    v_i = jnp.sum(m_c * wni_m[...], axis=0, keepdims=True)
    a_u = jnp.sum(ue_c[...] * wnu_h[...], axis=0, keepdims=True) + bnu_r[...]
    v_u = jnp.sum(m_cb * wnu_m[...], axis=0, keepdims=True)
    deg_a = jnp.sum(p_a[...], axis=0, keepdims=True)
    deg_b = jnp.sum(p_b[...], axis=0, keepdims=True)
    dn = (((0,), (0,)), ((), ()))
    out_i[...] = jax.nn.relu(
        a_i + lax.dot_general(deg_a, v_i, dn,
                              preferred_element_type=jnp.float32))
    out_u[...] = jax.nn.relu(
        a_u + lax.dot_general(deg_b, v_u, dn,
                              preferred_element_type=jnp.float32))


_BLK = 2560


def _const_spec(shape):
    return pl.BlockSpec(shape, lambda i: (0,) * len(shape))


_tc_call = pl.pallas_call(
    _tc_body,
    grid=(_NPAD // _BLK,),
    in_specs=[
        _const_spec((1, _D)), _const_spec((1, _D)),
        _const_spec((_D, 1)), _const_spec((_D, 1)),
        _const_spec((_D, _D)), _const_spec((_D, 1)),
        _const_spec((_D, _D)), _const_spec((_D, 1)),
        _const_spec((_D, _D)), _const_spec((_D, _D)), _const_spec((1, _D)),
        _const_spec((_D, _D)), _const_spec((_D, _D)), _const_spec((1, _D)),
        pl.BlockSpec((2 * _NS, _BLK), lambda i: (0, i)),
        pl.BlockSpec((2 * _NS, _BLK), lambda i: (0, i)),
    ],
    out_specs=(
        pl.BlockSpec((_BLK, _D), lambda i: (i, 0)),
        pl.BlockSpec((_BLK, _D), lambda i: (i, 0)),
    ),
    out_shape=(
        jax.ShapeDtypeStruct((_NU, _D), jnp.float32),
        jax.ShapeDtypeStruct((_NI, _D), jnp.float32),
    ),
)


@jax.jit
def kernel(clicks_src, clicks_dst, clicked_by_src, clicked_by_dst,
           user_emb, item_emb,
           W_clicks, b_clicks, W_clicked_by, b_clicked_by,
           Wn_user, bn_user, Wn_item, bn_item):
    del clicks_src, clicked_by_src
    p_a, p_b = _sc_degree_hist()(clicks_dst.astype(jnp.int32),
                                 clicked_by_dst.astype(jnp.int32))

    out_u, out_i = _tc_call(
        user_emb, item_emb,
        user_emb.reshape(_D, 1), item_emb.reshape(_D, 1),
        W_clicks.T, b_clicks.reshape(_D, 1),
        W_clicked_by.T, b_clicked_by.reshape(_D, 1),
        Wn_user[:_D], Wn_user[_D:], bn_user.reshape(1, _D),
        Wn_item[:_D], Wn_item[_D:], bn_item.reshape(1, _D),
        p_a, p_b)
    return (out_u, out_i)
